# baseline (speedup 1.0000x reference)
import functools

import jax
import jax.numpy as jnp
from jax import lax
from jax.experimental import pallas as pl
from jax.experimental.pallas import tpu as pltpu
from jax.experimental.pallas import tpu_sc as plsc

_BLK = 256


def _cumsum_rows(x):
    t = x.shape[0]
    c = x
    sh = 1
    while sh < t:
        z = jnp.zeros((sh, x.shape[1]), x.dtype)
        c = c + jnp.concatenate([z, c[:-sh]], axis=0)
        sh *= 2
    return c


def _router_meta_body(x_ref, gw_ref, tw0_ref, tw1_ref, pos0_ref, pos1_ref,
                      bexp_ref, nact_ref, *, n_exp, blk, nblk):
    lg = lax.dot_general(x_ref[...], gw_ref[...], (((1,), (1,)), ((), ())),
                         preferred_element_type=jnp.float32)
    m = jnp.max(lg, axis=1, keepdims=True)
    s = jnp.exp(lg - m)
    ii = lax.broadcasted_iota(jnp.int32, s.shape, 1)
    v1 = jnp.max(s, axis=1, keepdims=True)
    i1 = jnp.min(jnp.where(s == v1, ii, n_exp), axis=1, keepdims=True)
    s2 = jnp.where(ii == i1, -1.0, s)
    v2 = jnp.max(s2, axis=1, keepdims=True)
    i2 = jnp.min(jnp.where(s2 == v2, ii, n_exp), axis=1, keepdims=True)
    tw0_ref[...] = v1 / (v1 + v2)
    tw1_ref[...] = v2 / (v1 + v2)

    eq0 = (ii == i1).astype(jnp.int32)
    eq1 = (ii == i2).astype(jnp.int32)
    c0 = _cumsum_rows(eq0)
    c1 = _cumsum_rows(eq1)
    cnt0 = c0[-1:, :]
    cnt = cnt0 + c1[-1:, :]
    padded = ((cnt + (blk - 1)) // blk) * blk
    lt = (lax.broadcasted_iota(jnp.int32, (n_exp, n_exp), 0)
          < lax.broadcasted_iota(jnp.int32, (n_exp, n_exp), 1)
          ).astype(jnp.float32)
    offs = lax.dot_general(padded.astype(jnp.float32), lt,
                           (((1,), (0,)), ((), ())),
                           preferred_element_type=jnp.float32).astype(jnp.int32)
    pos0_ref[...] = jnp.sum(eq0 * (offs + c0 - 1), axis=1, keepdims=True)
    pos1_ref[...] = jnp.sum(eq1 * (offs + cnt0 + c1 - 1), axis=1, keepdims=True)

    ends = (offs + padded).astype(jnp.int32)
    bstart = lax.broadcasted_iota(jnp.int32, (nblk, 1), 0) * blk
    bexp = jnp.sum((bstart >= ends).astype(jnp.int32), axis=1, keepdims=True)
    bexp_ref[...] = jnp.minimum(bexp, n_exp - 1)
    nact_ref[...] = jnp.sum(padded, axis=1, keepdims=True) // blk


def _router_meta(x32, gate_w, nblk):
    t, _ = x32.shape
    n_exp = gate_w.shape[0]
    return pl.pallas_call(
        functools.partial(_router_meta_body, n_exp=n_exp, blk=_BLK, nblk=nblk),
        out_shape=[
            jax.ShapeDtypeStruct((t, 1), jnp.float32),
            jax.ShapeDtypeStruct((t, 1), jnp.float32),
            jax.ShapeDtypeStruct((t, 1), jnp.int32),
            jax.ShapeDtypeStruct((t, 1), jnp.int32),
            jax.ShapeDtypeStruct((nblk, 1), jnp.int32),
            jax.ShapeDtypeStruct((1, 1), jnp.int32),
        ],
    )(x32, gate_w)


def _sc_scatter_rows(xp, pos2, npad):
    t, w = xp.shape
    nw = 32
    cpw = t // nw
    mesh = plsc.VectorSubcoreMesh(core_axis_name="c", subcore_axis_name="s")

    @functools.partial(
        pl.kernel, mesh=mesh,
        out_type=jax.ShapeDtypeStruct((npad, w), jnp.int32),
        scratch_types=[
            pltpu.VMEM((cpw,), jnp.int32),
            pltpu.VMEM((cpw, w), jnp.int32),
            pltpu.SemaphoreType.DMA,
        ],
    )
    def k(x_hbm, pos_hbm, xs_hbm, idx_v, rows_v, sem):
        wid = lax.axis_index("s") * 2 + lax.axis_index("c")
        base = wid * cpw
        pltpu.sync_copy(x_hbm.at[pl.ds(base, cpw)], rows_v)
        pltpu.sync_copy(pos_hbm.at[0, pl.ds(base, cpw)], idx_v)
        pltpu.async_copy(rows_v, xs_hbm.at[idx_v], sem).wait()
        pltpu.sync_copy(pos_hbm.at[1, pl.ds(base, cpw)], idx_v)
        pltpu.async_copy(rows_v, xs_hbm.at[idx_v], sem).wait()

    return k(xp, pos2)


def _sc_gather_rows(ys, pos2):
    _, d = ys.shape
    _, t = pos2.shape
    nw = 32
    cpw = t // nw
    mesh = plsc.VectorSubcoreMesh(core_axis_name="c", subcore_axis_name="s")

    @functools.partial(
        pl.kernel, mesh=mesh,
        out_type=jax.ShapeDtypeStruct((2, t, d), jnp.float32),
        scratch_types=[
            pltpu.VMEM((cpw,), jnp.int32),
            pltpu.VMEM((cpw, d), jnp.float32),
            pltpu.SemaphoreType.DMA,
        ],
    )
    def k(ys_hbm, pos_hbm, g_hbm, idx_v, rows_v, sem):
        wid = lax.axis_index("s") * 2 + lax.axis_index("c")
        base = wid * cpw
        pltpu.sync_copy(pos_hbm.at[0, pl.ds(base, cpw)], idx_v)
        pltpu.async_copy(ys_hbm.at[idx_v], rows_v, sem).wait()
        pltpu.sync_copy(rows_v, g_hbm.at[0, pl.ds(base, cpw)])
        pltpu.sync_copy(pos_hbm.at[1, pl.ds(base, cpw)], idx_v)
        pltpu.async_copy(ys_hbm.at[idx_v], rows_v, sem).wait()
        pltpu.sync_copy(rows_v, g_hbm.at[1, pl.ds(base, cpw)])

    return k(ys, pos2)


def _grouped_glu_body(bexp_s, nact_s, xs_ref, w1_ref, w3_ref, w2_ref,
                      out_ref, acc_ref, *, nf):
    f = pl.program_id(0)
    a = pl.program_id(1)
    na = nact_s[0]

    @pl.when(a < na)
    def _():
        xb = xs_ref[...]
        w1b = w1_ref[0].astype(jnp.bfloat16)
        w3b = w3_ref[0].astype(jnp.bfloat16)
        t1 = lax.dot_general(xb, w1b, (((1,), (1,)), ((), ())),
                             preferred_element_type=jnp.float32)
        t3 = lax.dot_general(xb, w3b, (((1,), (1,)), ((), ())),
                             preferred_element_type=jnp.float32)
        h = (t1 * jax.nn.sigmoid(t1)) * t3
        w2b = w2_ref[0].astype(jnp.bfloat16)
        o = lax.dot_general(h.astype(jnp.bfloat16), w2b,
                            (((1,), (1,)), ((), ())),
                            preferred_element_type=jnp.float32)

        @pl.when(f == 0)
        def _():
            acc_ref[a] = o

        @pl.when(f != 0)
        def _():
            acc_ref[a] += o

        @pl.when(f == nf - 1)
        def _():
            out_ref[...] = acc_ref[a]


def _grouped_glu(xs, w1, w3, w2, bexp, nact, f_tile):
    npad, d = xs.shape
    n_exp, dff, _ = w1.shape
    nf = dff // f_tile
    nblk = npad // _BLK

    def amap(a, nact_s):
        return jnp.minimum(a, nact_s[0] - 1)

    grid_spec = pltpu.PrefetchScalarGridSpec(
        num_scalar_prefetch=2,
        grid=(nf, nblk),
        in_specs=[
            pl.BlockSpec((_BLK, d), lambda f, a, be, na: (amap(a, na), 0)),
            pl.BlockSpec((1, f_tile, d),
                         lambda f, a, be, na: (be[amap(a, na)], f, 0)),
            pl.BlockSpec((1, f_tile, d),
                         lambda f, a, be, na: (be[amap(a, na)], f, 0)),
            pl.BlockSpec((1, d, f_tile),
                         lambda f, a, be, na: (be[amap(a, na)], 0, f)),
        ],
        out_specs=pl.BlockSpec(
            (_BLK, d),
            lambda f, a, be, na: (jnp.where(f == nf - 1, amap(a, na), 0), 0)),
        scratch_shapes=[pltpu.VMEM((nblk, _BLK, d), jnp.float32)],
    )
    return pl.pallas_call(
        functools.partial(_grouped_glu_body, nf=nf),
        grid_spec=grid_spec,
        out_shape=jax.ShapeDtypeStruct((npad, d), jnp.float32),
        compiler_params=pltpu.CompilerParams(
            dimension_semantics=("arbitrary", "arbitrary")),
    )(bexp, nact, xs, w1, w3, w2)


def _combine_body(g_ref, tw0_ref, tw1_ref, out_ref):
    out_ref[...] = g_ref[0] * tw0_ref[...] + g_ref[1] * tw1_ref[...]


def _combine(g, tw0, tw1):
    _, t, d = g.shape
    return pl.pallas_call(
        _combine_body,
        out_shape=jax.ShapeDtypeStruct((t, d), jnp.float32),
    )(g, tw0, tw1)


def _moe_pipeline(x32, gate_w, w1, w3, w2, f_tile=512):
    t, d = x32.shape
    n_exp = gate_w.shape[0]
    nblk = (2 * t + (n_exp - 1) * _BLK) // _BLK
    npad = nblk * _BLK
    tw0, tw1, pos0, pos1, bexp2, nact2 = _router_meta(x32, gate_w, nblk)
    pos2 = jnp.stack([pos0[:, 0], pos1[:, 0]])
    xb = x32.astype(jnp.bfloat16)
    xp = lax.bitcast_convert_type(xb.reshape(t, d // 2, 2), jnp.int32)
    xsp = _sc_scatter_rows(xp, pos2, npad)
    xs = lax.bitcast_convert_type(xsp, jnp.bfloat16).reshape(npad, d)
    ys = _grouped_glu(xs, w1, w3, w2, bexp2[:, 0], nact2[0], f_tile)
    g = _sc_gather_rows(ys, pos2)
    return _combine(g, tw0, tw1)


def kernel(hidden_states, gate_w, w1, w3, w2):
    b, s, d = hidden_states.shape
    x32 = hidden_states.reshape(b * s, d)
    out = _moe_pipeline(x32, gate_w, w1, w3, w2)
    return out.reshape(b, s, d)


# speedup vs baseline: 1.3432x; 1.3432x over previous
import functools

import jax
import jax.numpy as jnp
from jax import lax
from jax.experimental import pallas as pl
from jax.experimental.pallas import tpu as pltpu
from jax.experimental.pallas import tpu_sc as plsc

_BLK = 256


def _cumsum_rows(x):
    t = x.shape[0]
    c = x
    sh = 1
    while sh < t:
        z = jnp.zeros((sh, x.shape[1]), x.dtype)
        c = c + jnp.concatenate([z, c[:-sh]], axis=0)
        sh *= 2
    return c


def _router_meta_body(x_ref, gw_ref, tw0_ref, tw1_ref, pos0_ref, pos1_ref,
                      bexp_ref, nact_ref, *, n_exp, blk, nblk):
    lg = lax.dot_general(x_ref[...], gw_ref[...], (((1,), (1,)), ((), ())),
                         preferred_element_type=jnp.float32)
    m = jnp.max(lg, axis=1, keepdims=True)
    s = jnp.exp(lg - m)
    ii = lax.broadcasted_iota(jnp.int32, s.shape, 1)
    v1 = jnp.max(s, axis=1, keepdims=True)
    i1 = jnp.min(jnp.where(s == v1, ii, n_exp), axis=1, keepdims=True)
    s2 = jnp.where(ii == i1, -1.0, s)
    v2 = jnp.max(s2, axis=1, keepdims=True)
    i2 = jnp.min(jnp.where(s2 == v2, ii, n_exp), axis=1, keepdims=True)
    tw0_ref[...] = v1 / (v1 + v2)
    tw1_ref[...] = v2 / (v1 + v2)

    eq0 = (ii == i1).astype(jnp.int32)
    eq1 = (ii == i2).astype(jnp.int32)
    c0 = _cumsum_rows(eq0)
    c1 = _cumsum_rows(eq1)
    cnt0 = c0[-1:, :]
    cnt = cnt0 + c1[-1:, :]
    padded = ((cnt + (blk - 1)) // blk) * blk
    lt = (lax.broadcasted_iota(jnp.int32, (n_exp, n_exp), 0)
          < lax.broadcasted_iota(jnp.int32, (n_exp, n_exp), 1)
          ).astype(jnp.float32)
    offs = lax.dot_general(padded.astype(jnp.float32), lt,
                           (((1,), (0,)), ((), ())),
                           preferred_element_type=jnp.float32).astype(jnp.int32)
    pos0_ref[...] = jnp.sum(eq0 * (offs + c0 - 1), axis=1, keepdims=True)
    pos1_ref[...] = jnp.sum(eq1 * (offs + cnt0 + c1 - 1), axis=1, keepdims=True)

    ends = (offs + padded).astype(jnp.int32)
    bstart = lax.broadcasted_iota(jnp.int32, (nblk, 1), 0) * blk
    bexp = jnp.sum((bstart >= ends).astype(jnp.int32), axis=1, keepdims=True)
    bexp_ref[...] = jnp.minimum(bexp, n_exp - 1)
    nact_ref[...] = jnp.sum(padded, axis=1, keepdims=True) // blk


def _router_meta(x32, gate_w, nblk):
    t, _ = x32.shape
    n_exp = gate_w.shape[0]
    return pl.pallas_call(
        functools.partial(_router_meta_body, n_exp=n_exp, blk=_BLK, nblk=nblk),
        out_shape=[
            jax.ShapeDtypeStruct((t, 1), jnp.float32),
            jax.ShapeDtypeStruct((t, 1), jnp.float32),
            jax.ShapeDtypeStruct((t, 1), jnp.int32),
            jax.ShapeDtypeStruct((t, 1), jnp.int32),
            jax.ShapeDtypeStruct((nblk, 1), jnp.int32),
            jax.ShapeDtypeStruct((1, 1), jnp.int32),
        ],
    )(x32, gate_w)


def _sc_scatter_rows(xp, pos2, npad):
    t, w = xp.shape
    nw = 32
    cpw = t // nw
    mesh = plsc.VectorSubcoreMesh(core_axis_name="c", subcore_axis_name="s")

    @functools.partial(
        pl.kernel, mesh=mesh,
        out_type=jax.ShapeDtypeStruct((npad, w), jnp.int32),
        scratch_types=[
            pltpu.VMEM((cpw,), jnp.int32),
            pltpu.VMEM((cpw, w), jnp.int32),
            pltpu.SemaphoreType.DMA,
        ],
    )
    def k(x_hbm, pos_hbm, xs_hbm, idx_v, rows_v, sem):
        wid = lax.axis_index("s") * 2 + lax.axis_index("c")
        base = wid * cpw
        pltpu.sync_copy(x_hbm.at[pl.ds(base, cpw)], rows_v)
        pltpu.sync_copy(pos_hbm.at[0, pl.ds(base, cpw)], idx_v)
        pltpu.async_copy(rows_v, xs_hbm.at[idx_v], sem).wait()
        pltpu.sync_copy(pos_hbm.at[1, pl.ds(base, cpw)], idx_v)
        pltpu.async_copy(rows_v, xs_hbm.at[idx_v], sem).wait()

    return k(xp, pos2)


def _sc_gather_rows(ys, pos2):
    _, d = ys.shape
    _, t = pos2.shape
    nw = 32
    cpw = t // nw
    mesh = plsc.VectorSubcoreMesh(core_axis_name="c", subcore_axis_name="s")

    @functools.partial(
        pl.kernel, mesh=mesh,
        out_type=jax.ShapeDtypeStruct((2, t, d), jnp.float32),
        scratch_types=[
            pltpu.VMEM((cpw,), jnp.int32),
            pltpu.VMEM((cpw, d), jnp.float32),
            pltpu.SemaphoreType.DMA,
        ],
    )
    def k(ys_hbm, pos_hbm, g_hbm, idx_v, rows_v, sem):
        wid = lax.axis_index("s") * 2 + lax.axis_index("c")
        base = wid * cpw
        pltpu.sync_copy(pos_hbm.at[0, pl.ds(base, cpw)], idx_v)
        pltpu.async_copy(ys_hbm.at[idx_v], rows_v, sem).wait()
        pltpu.sync_copy(rows_v, g_hbm.at[0, pl.ds(base, cpw)])
        pltpu.sync_copy(pos_hbm.at[1, pl.ds(base, cpw)], idx_v)
        pltpu.async_copy(ys_hbm.at[idx_v], rows_v, sem).wait()
        pltpu.sync_copy(rows_v, g_hbm.at[1, pl.ds(base, cpw)])

    return k(ys, pos2)


def _grouped_glu_body(bexp_s, nact_s, xs_ref, w1_ref, w3_ref, w2_ref,
                      out_ref, acc_ref, *, nf):
    f = pl.program_id(0)
    a = pl.program_id(1)
    na = nact_s[0]

    @pl.when(a < na)
    def _():
        xb = xs_ref[...].astype(jnp.bfloat16)
        w1b = w1_ref[0].astype(jnp.bfloat16)
        w3b = w3_ref[0].astype(jnp.bfloat16)
        t1 = lax.dot_general(xb, w1b, (((1,), (1,)), ((), ())),
                             preferred_element_type=jnp.float32)
        t3 = lax.dot_general(xb, w3b, (((1,), (1,)), ((), ())),
                             preferred_element_type=jnp.float32)
        h = (t1 * jax.nn.sigmoid(t1)) * t3
        w2b = w2_ref[0].astype(jnp.bfloat16)
        o = lax.dot_general(h.astype(jnp.bfloat16), w2b,
                            (((1,), (1,)), ((), ())),
                            preferred_element_type=jnp.float32)

        @pl.when(f == 0)
        def _():
            acc_ref[a] = o

        @pl.when(f != 0)
        def _():
            acc_ref[a] += o

        @pl.when(f == nf - 1)
        def _():
            out_ref[...] = acc_ref[a]


def _grouped_glu(xs, w1, w3, w2, bexp, nact, f_tile):
    npad, d = xs.shape
    n_exp, dff, _ = w1.shape
    nf = dff // f_tile
    nblk = npad // _BLK

    def amap(a, nact_s):
        return jnp.minimum(a, nact_s[0] - 1)

    grid_spec = pltpu.PrefetchScalarGridSpec(
        num_scalar_prefetch=2,
        grid=(nf, nblk),
        in_specs=[
            pl.BlockSpec((_BLK, d), lambda f, a, be, na: (amap(a, na), 0)),
            pl.BlockSpec((1, f_tile, d),
                         lambda f, a, be, na: (be[amap(a, na)], f, 0)),
            pl.BlockSpec((1, f_tile, d),
                         lambda f, a, be, na: (be[amap(a, na)], f, 0)),
            pl.BlockSpec((1, d, f_tile),
                         lambda f, a, be, na: (be[amap(a, na)], 0, f)),
        ],
        out_specs=pl.BlockSpec(
            (_BLK, d),
            lambda f, a, be, na: (jnp.where(f == nf - 1, amap(a, na), 0), 0)),
        scratch_shapes=[pltpu.VMEM((nblk, _BLK, d), jnp.float32)],
    )
    return pl.pallas_call(
        functools.partial(_grouped_glu_body, nf=nf),
        grid_spec=grid_spec,
        out_shape=jax.ShapeDtypeStruct((npad, d), jnp.float32),
        compiler_params=pltpu.CompilerParams(
            dimension_semantics=("arbitrary", "arbitrary")),
    )(bexp, nact, xs, w1, w3, w2)


def _combine_body(g_ref, tw0_ref, tw1_ref, out_ref):
    out_ref[...] = g_ref[0] * tw0_ref[...] + g_ref[1] * tw1_ref[...]


def _combine(g, tw0, tw1):
    _, t, d = g.shape
    return pl.pallas_call(
        _combine_body,
        out_shape=jax.ShapeDtypeStruct((t, d), jnp.float32),
    )(g, tw0, tw1)


def _moe_pipeline(x32, gate_w, w1, w3, w2, f_tile=512):
    t, d = x32.shape
    n_exp = gate_w.shape[0]
    nblk = (2 * t + (n_exp - 1) * _BLK) // _BLK
    npad = nblk * _BLK
    tw0, tw1, pos0, pos1, bexp2, nact2 = _router_meta(x32, gate_w, nblk)
    pos2 = jnp.stack([pos0[:, 0], pos1[:, 0]])
    xp = lax.bitcast_convert_type(x32, jnp.int32)
    xsp = _sc_scatter_rows(xp, pos2, npad)
    xs = lax.bitcast_convert_type(xsp, jnp.float32)
    ys = _grouped_glu(xs, w1, w3, w2, bexp2[:, 0], nact2[0], f_tile)
    g = _sc_gather_rows(ys, pos2)
    return _combine(g, tw0, tw1)


def kernel(hidden_states, gate_w, w1, w3, w2):
    b, s, d = hidden_states.shape
    x32 = hidden_states.reshape(b * s, d)
    out = _moe_pipeline(x32, gate_w, w1, w3, w2)
    return out.reshape(b, s, d)


# speedup vs baseline: 1.4316x; 1.0658x over previous
import functools

import jax
import jax.numpy as jnp
from jax import lax
from jax.experimental import pallas as pl
from jax.experimental.pallas import tpu as pltpu
from jax.experimental.pallas import tpu_sc as plsc

_BLK = 256


def _cumsum_rows(x):
    t = x.shape[0]
    c = x
    sh = 1
    while sh < t:
        z = jnp.zeros((sh, x.shape[1]), x.dtype)
        c = c + jnp.concatenate([z, c[:-sh]], axis=0)
        sh *= 2
    return c


def _router_meta_body(x_ref, gw_ref, tw0_ref, tw1_ref, pos0_ref, pos1_ref,
                      bexp_ref, nact_ref, *, n_exp, blk, nblk):
    lg = lax.dot_general(x_ref[...], gw_ref[...], (((1,), (1,)), ((), ())),
                         preferred_element_type=jnp.float32)
    m = jnp.max(lg, axis=1, keepdims=True)
    s = jnp.exp(lg - m)
    ii = lax.broadcasted_iota(jnp.int32, s.shape, 1)
    v1 = jnp.max(s, axis=1, keepdims=True)
    i1 = jnp.min(jnp.where(s == v1, ii, n_exp), axis=1, keepdims=True)
    s2 = jnp.where(ii == i1, -1.0, s)
    v2 = jnp.max(s2, axis=1, keepdims=True)
    i2 = jnp.min(jnp.where(s2 == v2, ii, n_exp), axis=1, keepdims=True)
    tw0_ref[...] = v1 / (v1 + v2)
    tw1_ref[...] = v2 / (v1 + v2)

    eq0 = (ii == i1).astype(jnp.int32)
    eq1 = (ii == i2).astype(jnp.int32)
    c0 = _cumsum_rows(eq0)
    c1 = _cumsum_rows(eq1)
    cnt0 = c0[-1:, :]
    cnt = cnt0 + c1[-1:, :]
    padded = ((cnt + (blk - 1)) // blk) * blk
    lt = (lax.broadcasted_iota(jnp.int32, (n_exp, n_exp), 0)
          < lax.broadcasted_iota(jnp.int32, (n_exp, n_exp), 1)
          ).astype(jnp.float32)
    offs = lax.dot_general(padded.astype(jnp.float32), lt,
                           (((1,), (0,)), ((), ())),
                           preferred_element_type=jnp.float32).astype(jnp.int32)
    pos0_ref[...] = jnp.sum(eq0 * (offs + c0 - 1), axis=1, keepdims=True)
    pos1_ref[...] = jnp.sum(eq1 * (offs + cnt0 + c1 - 1), axis=1, keepdims=True)

    ends = (offs + padded).astype(jnp.int32)
    bstart = lax.broadcasted_iota(jnp.int32, (nblk, 1), 0) * blk
    bexp = jnp.sum((bstart >= ends).astype(jnp.int32), axis=1, keepdims=True)
    bexp_ref[...] = jnp.minimum(bexp, n_exp - 1)
    nact_ref[...] = jnp.sum(padded, axis=1, keepdims=True) // blk


def _router_meta(x32, gate_w, nblk):
    t, _ = x32.shape
    n_exp = gate_w.shape[0]
    return pl.pallas_call(
        functools.partial(_router_meta_body, n_exp=n_exp, blk=_BLK, nblk=nblk),
        out_shape=[
            jax.ShapeDtypeStruct((t, 1), jnp.float32),
            jax.ShapeDtypeStruct((t, 1), jnp.float32),
            jax.ShapeDtypeStruct((t, 1), jnp.int32),
            jax.ShapeDtypeStruct((t, 1), jnp.int32),
            jax.ShapeDtypeStruct((nblk, 1), jnp.int32),
            jax.ShapeDtypeStruct((1, 1), jnp.int32),
        ],
    )(x32, gate_w)


def _sc_scatter_rows(xp, pos2, npad):
    t, w = xp.shape
    nw = 32
    cpw = t // nw
    mesh = plsc.VectorSubcoreMesh(core_axis_name="c", subcore_axis_name="s")

    @functools.partial(
        pl.kernel, mesh=mesh,
        out_type=jax.ShapeDtypeStruct((npad, w), jnp.float32),
        scratch_types=[
            pltpu.VMEM((cpw,), jnp.int32),
            pltpu.VMEM((cpw, w), jnp.float32),
            pltpu.SemaphoreType.DMA,
        ],
    )
    def k(x_hbm, pos_hbm, xs_hbm, idx_v, rows_v, sem):
        wid = lax.axis_index("s") * 2 + lax.axis_index("c")
        base = wid * cpw
        pltpu.sync_copy(x_hbm.at[pl.ds(base, cpw)], rows_v)
        pltpu.sync_copy(pos_hbm.at[0, pl.ds(base, cpw)], idx_v)
        pltpu.async_copy(rows_v, xs_hbm.at[idx_v], sem).wait()
        pltpu.sync_copy(pos_hbm.at[1, pl.ds(base, cpw)], idx_v)
        pltpu.async_copy(rows_v, xs_hbm.at[idx_v], sem).wait()

    return k(xp, pos2)


def _sc_gather_rows(ys, pos2):
    _, d = ys.shape
    _, t = pos2.shape
    nw = 32
    cpw = t // nw
    mesh = plsc.VectorSubcoreMesh(core_axis_name="c", subcore_axis_name="s")

    @functools.partial(
        pl.kernel, mesh=mesh,
        out_type=jax.ShapeDtypeStruct((2, t, d), jnp.float32),
        scratch_types=[
            pltpu.VMEM((cpw,), jnp.int32),
            pltpu.VMEM((cpw, d), jnp.float32),
            pltpu.SemaphoreType.DMA,
        ],
    )
    def k(ys_hbm, pos_hbm, g_hbm, idx_v, rows_v, sem):
        wid = lax.axis_index("s") * 2 + lax.axis_index("c")
        base = wid * cpw
        pltpu.sync_copy(pos_hbm.at[0, pl.ds(base, cpw)], idx_v)
        pltpu.async_copy(ys_hbm.at[idx_v], rows_v, sem).wait()
        pltpu.sync_copy(rows_v, g_hbm.at[0, pl.ds(base, cpw)])
        pltpu.sync_copy(pos_hbm.at[1, pl.ds(base, cpw)], idx_v)
        pltpu.async_copy(ys_hbm.at[idx_v], rows_v, sem).wait()
        pltpu.sync_copy(rows_v, g_hbm.at[1, pl.ds(base, cpw)])

    return k(ys, pos2)


def _grouped_glu_body(bexp_s, nact_s, xs_ref, w1_ref, w3_ref, w2_ref,
                      out_ref, acc_ref, *, nf):
    f = pl.program_id(0)
    a = pl.program_id(1)
    na = nact_s[0]

    @pl.when(a < na)
    def _():
        xb = xs_ref[...].astype(jnp.bfloat16)
        w1b = w1_ref[0].astype(jnp.bfloat16)
        w3b = w3_ref[0].astype(jnp.bfloat16)
        t1 = lax.dot_general(xb, w1b, (((1,), (1,)), ((), ())),
                             preferred_element_type=jnp.float32)
        t3 = lax.dot_general(xb, w3b, (((1,), (1,)), ((), ())),
                             preferred_element_type=jnp.float32)
        h = (t1 * jax.nn.sigmoid(t1)) * t3
        w2b = w2_ref[0].astype(jnp.bfloat16)
        o = lax.dot_general(h.astype(jnp.bfloat16), w2b,
                            (((1,), (1,)), ((), ())),
                            preferred_element_type=jnp.float32)

        @pl.when(f == 0)
        def _():
            acc_ref[a] = o

        @pl.when(f != 0)
        def _():
            acc_ref[a] += o

        @pl.when(f == nf - 1)
        def _():
            out_ref[...] = acc_ref[a]


def _grouped_glu(xs, w1, w3, w2, bexp, nact, f_tile):
    npad, d = xs.shape
    n_exp, dff, _ = w1.shape
    nf = dff // f_tile
    nblk = npad // _BLK

    def amap(a, nact_s):
        return jnp.minimum(a, nact_s[0] - 1)

    grid_spec = pltpu.PrefetchScalarGridSpec(
        num_scalar_prefetch=2,
        grid=(nf, nblk),
        in_specs=[
            pl.BlockSpec((_BLK, d), lambda f, a, be, na: (amap(a, na), 0)),
            pl.BlockSpec((1, f_tile, d),
                         lambda f, a, be, na: (be[amap(a, na)], f, 0)),
            pl.BlockSpec((1, f_tile, d),
                         lambda f, a, be, na: (be[amap(a, na)], f, 0)),
            pl.BlockSpec((1, d, f_tile),
                         lambda f, a, be, na: (be[amap(a, na)], 0, f)),
        ],
        out_specs=pl.BlockSpec(
            (_BLK, d),
            lambda f, a, be, na: (jnp.where(f == nf - 1, amap(a, na), 0), 0)),
        scratch_shapes=[pltpu.VMEM((nblk, _BLK, d), jnp.float32)],
    )
    return pl.pallas_call(
        functools.partial(_grouped_glu_body, nf=nf),
        grid_spec=grid_spec,
        out_shape=jax.ShapeDtypeStruct((npad, d), jnp.float32),
        compiler_params=pltpu.CompilerParams(
            dimension_semantics=("arbitrary", "arbitrary")),
    )(bexp, nact, xs, w1, w3, w2)


def _combine_body(g_ref, tw0_ref, tw1_ref, out_ref):
    out_ref[...] = g_ref[0] * tw0_ref[...] + g_ref[1] * tw1_ref[...]


def _combine(g, tw0, tw1):
    _, t, d = g.shape
    return pl.pallas_call(
        _combine_body,
        out_shape=jax.ShapeDtypeStruct((t, d), jnp.float32),
    )(g, tw0, tw1)


def _moe_pipeline(x32, gate_w, w1, w3, w2, f_tile=512):
    t, d = x32.shape
    n_exp = gate_w.shape[0]
    nblk = (2 * t + (n_exp - 1) * _BLK) // _BLK
    npad = nblk * _BLK
    tw0, tw1, pos0, pos1, bexp2, nact2 = _router_meta(x32, gate_w, nblk)
    pos2 = jnp.stack([pos0[:, 0], pos1[:, 0]])
    xs = _sc_scatter_rows(x32, pos2, npad)
    ys = _grouped_glu(xs, w1, w3, w2, bexp2[:, 0], nact2[0], f_tile)
    g = _sc_gather_rows(ys, pos2)
    return _combine(g, tw0, tw1)


def kernel(hidden_states, gate_w, w1, w3, w2):
    b, s, d = hidden_states.shape
    x32 = hidden_states.reshape(b * s, d)
    out = _moe_pipeline(x32, gate_w, w1, w3, w2)
    return out.reshape(b, s, d)


# speedup vs baseline: 1.6562x; 1.1569x over previous
import functools

import jax
import jax.numpy as jnp
from jax import lax
from jax.experimental import pallas as pl
from jax.experimental.pallas import tpu as pltpu
from jax.experimental.pallas import tpu_sc as plsc

_BLK = 256


def _cumsum_rows(x):
    t = x.shape[0]
    c = x
    sh = 1
    while sh < t:
        z = jnp.zeros((sh, x.shape[1]), x.dtype)
        c = c + jnp.concatenate([z, c[:-sh]], axis=0)
        sh *= 2
    return c


def _router_meta_body(x_ref, gw_ref, tw0_ref, tw1_ref, pos0_ref, pos1_ref,
                      bexp_ref, nact_ref, *, n_exp, blk, nblk):
    lg = lax.dot_general(x_ref[...], gw_ref[...], (((1,), (1,)), ((), ())),
                         preferred_element_type=jnp.float32)
    m = jnp.max(lg, axis=1, keepdims=True)
    s = jnp.exp(lg - m)
    ii = lax.broadcasted_iota(jnp.int32, s.shape, 1)
    v1 = jnp.max(s, axis=1, keepdims=True)
    i1 = jnp.min(jnp.where(s == v1, ii, n_exp), axis=1, keepdims=True)
    s2 = jnp.where(ii == i1, -1.0, s)
    v2 = jnp.max(s2, axis=1, keepdims=True)
    i2 = jnp.min(jnp.where(s2 == v2, ii, n_exp), axis=1, keepdims=True)
    tw0_ref[...] = v1 / (v1 + v2)
    tw1_ref[...] = v2 / (v1 + v2)

    eq0 = (ii == i1).astype(jnp.int32)
    eq1 = (ii == i2).astype(jnp.int32)
    c0 = _cumsum_rows(eq0)
    c1 = _cumsum_rows(eq1)
    cnt0 = c0[-1:, :]
    cnt = cnt0 + c1[-1:, :]
    padded = ((cnt + (blk - 1)) // blk) * blk
    lt = (lax.broadcasted_iota(jnp.int32, (n_exp, n_exp), 0)
          < lax.broadcasted_iota(jnp.int32, (n_exp, n_exp), 1)
          ).astype(jnp.float32)
    offs = lax.dot_general(padded.astype(jnp.float32), lt,
                           (((1,), (0,)), ((), ())),
                           preferred_element_type=jnp.float32).astype(jnp.int32)
    pos0_ref[...] = jnp.sum(eq0 * (offs + c0 - 1), axis=1, keepdims=True)
    pos1_ref[...] = jnp.sum(eq1 * (offs + cnt0 + c1 - 1), axis=1, keepdims=True)

    ends = (offs + padded).astype(jnp.int32)
    bstart = lax.broadcasted_iota(jnp.int32, (nblk, 1), 0) * blk
    bexp = jnp.sum((bstart >= ends).astype(jnp.int32), axis=1, keepdims=True)
    bexp_ref[...] = jnp.minimum(bexp, n_exp - 1)
    nact_ref[...] = jnp.sum(padded, axis=1, keepdims=True) // blk


def _router_meta(x32, gate_w, nblk):
    t, _ = x32.shape
    n_exp = gate_w.shape[0]
    return pl.pallas_call(
        functools.partial(_router_meta_body, n_exp=n_exp, blk=_BLK, nblk=nblk),
        out_shape=[
            jax.ShapeDtypeStruct((t, 1), jnp.float32),
            jax.ShapeDtypeStruct((t, 1), jnp.float32),
            jax.ShapeDtypeStruct((t, 1), jnp.int32),
            jax.ShapeDtypeStruct((t, 1), jnp.int32),
            jax.ShapeDtypeStruct((nblk, 1), jnp.int32),
            jax.ShapeDtypeStruct((1, 1), jnp.int32),
        ],
    )(x32, gate_w)


def _sc_scatter_rows(xp, pos2, npad):
    t, w = xp.shape
    nw = 32
    cpw = t // nw
    mesh = plsc.VectorSubcoreMesh(core_axis_name="c", subcore_axis_name="s")

    @functools.partial(
        pl.kernel, mesh=mesh,
        out_type=jax.ShapeDtypeStruct((npad, w), jnp.float32),
        scratch_types=[
            pltpu.VMEM((cpw,), jnp.int32),
            pltpu.VMEM((cpw, w), jnp.float32),
            pltpu.SemaphoreType.DMA,
        ],
    )
    def k(x_hbm, pos_hbm, xs_hbm, idx_v, rows_v, sem):
        wid = lax.axis_index("s") * 2 + lax.axis_index("c")
        base = wid * cpw
        pltpu.sync_copy(x_hbm.at[pl.ds(base, cpw)], rows_v)
        pltpu.sync_copy(pos_hbm.at[0, pl.ds(base, cpw)], idx_v)
        pltpu.async_copy(rows_v, xs_hbm.at[idx_v], sem).wait()
        pltpu.sync_copy(pos_hbm.at[1, pl.ds(base, cpw)], idx_v)
        pltpu.async_copy(rows_v, xs_hbm.at[idx_v], sem).wait()

    return k(xp, pos2)


def _sc_gather_rows(ys, pos2):
    _, d = ys.shape
    _, t = pos2.shape
    nw = 32
    cpw = t // nw
    mesh = plsc.VectorSubcoreMesh(core_axis_name="c", subcore_axis_name="s")

    @functools.partial(
        pl.kernel, mesh=mesh,
        out_type=jax.ShapeDtypeStruct((2, t, d), jnp.float32),
        scratch_types=[
            pltpu.VMEM((cpw,), jnp.int32),
            pltpu.VMEM((cpw, d), jnp.float32),
            pltpu.SemaphoreType.DMA,
        ],
    )
    def k(ys_hbm, pos_hbm, g_hbm, idx_v, rows_v, sem):
        wid = lax.axis_index("s") * 2 + lax.axis_index("c")
        base = wid * cpw
        pltpu.sync_copy(pos_hbm.at[0, pl.ds(base, cpw)], idx_v)
        pltpu.async_copy(ys_hbm.at[idx_v], rows_v, sem).wait()
        pltpu.sync_copy(rows_v, g_hbm.at[0, pl.ds(base, cpw)])
        pltpu.sync_copy(pos_hbm.at[1, pl.ds(base, cpw)], idx_v)
        pltpu.async_copy(ys_hbm.at[idx_v], rows_v, sem).wait()
        pltpu.sync_copy(rows_v, g_hbm.at[1, pl.ds(base, cpw)])

    return k(ys, pos2)


def _grouped_glu_body(bexp_s, nact_s, xs_ref, w1_ref, w3_ref, w2_ref,
                      out_ref, acc_ref, *, nf):
    f = pl.program_id(0)
    a = pl.program_id(1)
    na = nact_s[0]

    @pl.when(a < na)
    def _():
        xb = xs_ref[...].astype(jnp.bfloat16)
        w1b = w1_ref[0].astype(jnp.bfloat16)
        w3b = w3_ref[0].astype(jnp.bfloat16)
        t1 = lax.dot_general(xb, w1b, (((1,), (1,)), ((), ())),
                             preferred_element_type=jnp.float32)
        t3 = lax.dot_general(xb, w3b, (((1,), (1,)), ((), ())),
                             preferred_element_type=jnp.float32)
        h = (t1 * jax.nn.sigmoid(t1)) * t3
        w2b = w2_ref[0].astype(jnp.bfloat16)
        o = lax.dot_general(h.astype(jnp.bfloat16), w2b,
                            (((1,), (1,)), ((), ())),
                            preferred_element_type=jnp.float32)

        @pl.when(f == 0)
        def _():
            acc_ref[a] = o

        @pl.when(f != 0)
        def _():
            acc_ref[a] += o

        @pl.when(f == nf - 1)
        def _():
            out_ref[...] = acc_ref[a]


def _grouped_glu(xs, w1, w3, w2, bexp, nact, f_tile):
    npad, d = xs.shape
    n_exp, dff, _ = w1.shape
    nf = dff // f_tile
    nblk = npad // _BLK

    def amap(a, nact_s):
        return jnp.minimum(a, nact_s[0] - 1)

    grid_spec = pltpu.PrefetchScalarGridSpec(
        num_scalar_prefetch=2,
        grid=(nf, nblk),
        in_specs=[
            pl.BlockSpec((_BLK, d), lambda f, a, be, na: (amap(a, na), 0)),
            pl.BlockSpec((1, f_tile, d),
                         lambda f, a, be, na: (be[amap(a, na)], f, 0)),
            pl.BlockSpec((1, f_tile, d),
                         lambda f, a, be, na: (be[amap(a, na)], f, 0)),
            pl.BlockSpec((1, d, f_tile),
                         lambda f, a, be, na: (be[amap(a, na)], 0, f)),
        ],
        out_specs=pl.BlockSpec(
            (_BLK, d),
            lambda f, a, be, na: (jnp.where(f == nf - 1, amap(a, na), 0), 0)),
        scratch_shapes=[pltpu.VMEM((nblk, _BLK, d), jnp.float32)],
    )
    return pl.pallas_call(
        functools.partial(_grouped_glu_body, nf=nf),
        grid_spec=grid_spec,
        out_shape=jax.ShapeDtypeStruct((npad, d), jnp.float32),
        compiler_params=pltpu.CompilerParams(
            dimension_semantics=("arbitrary", "arbitrary")),
    )(bexp, nact, xs, w1, w3, w2)


def _combine_body(g_ref, tw0_ref, tw1_ref, out_ref):
    out_ref[...] = g_ref[0] * tw0_ref[...] + g_ref[1] * tw1_ref[...]


def _combine(g, tw0, tw1):
    _, t, d = g.shape
    return pl.pallas_call(
        _combine_body,
        out_shape=jax.ShapeDtypeStruct((t, d), jnp.float32),
    )(g, tw0, tw1)


def _moe_pipeline(x32, gate_w, w1, w3, w2, f_tile=512):
    t, d = x32.shape
    n_exp = gate_w.shape[0]
    nblk = (2 * t + (n_exp - 1) * _BLK) // _BLK
    npad = nblk * _BLK
    tw0, tw1, pos0, pos1, bexp2, nact2 = _router_meta(x32, gate_w, nblk)
    pos2 = jnp.stack([pos0[:, 0], pos1[:, 0]])
    xs = _sc_scatter_rows(x32, pos2, npad)
    ys = _grouped_glu(xs, w1, w3, w2, bexp2[:, 0], nact2[0], f_tile)
    g = _sc_gather_rows(ys, pos2)
    return _combine(g, tw0, tw1)


def kernel(hidden_states, gate_w, w1, w3, w2):
    b, s, d = hidden_states.shape
    x32 = hidden_states.reshape(b * s, d)
    out = _moe_pipeline(x32, gate_w, w1, w3, w2, f_tile=896)
    return out.reshape(b, s, d)


# speedup vs baseline: 1.6638x; 1.0046x over previous
import functools

import jax
import jax.numpy as jnp
from jax import lax
from jax.experimental import pallas as pl
from jax.experimental.pallas import tpu as pltpu
from jax.experimental.pallas import tpu_sc as plsc

_BLK = 256


def _cumsum_rows(x):
    t = x.shape[0]
    c = x
    sh = 1
    while sh < t:
        z = jnp.zeros((sh, x.shape[1]), x.dtype)
        c = c + jnp.concatenate([z, c[:-sh]], axis=0)
        sh *= 2
    return c


def _router_meta_body(x_ref, gw_ref, tw16_ref, pos0_ref, pos1_ref,
                      bexp_ref, nact_ref, *, n_exp, blk, nblk):
    lg = lax.dot_general(x_ref[...], gw_ref[...], (((1,), (1,)), ((), ())),
                         preferred_element_type=jnp.float32)
    m = jnp.max(lg, axis=1, keepdims=True)
    s = jnp.exp(lg - m)
    ii = lax.broadcasted_iota(jnp.int32, s.shape, 1)
    v1 = jnp.max(s, axis=1, keepdims=True)
    i1 = jnp.min(jnp.where(s == v1, ii, n_exp), axis=1, keepdims=True)
    s2 = jnp.where(ii == i1, -1.0, s)
    v2 = jnp.max(s2, axis=1, keepdims=True)
    i2 = jnp.min(jnp.where(s2 == v2, ii, n_exp), axis=1, keepdims=True)
    t = s.shape[0]
    tw16_ref[...] = jnp.concatenate(
        [jnp.broadcast_to(v1 / (v1 + v2), (t, 128)),
         jnp.broadcast_to(v2 / (v1 + v2), (t, 128))], axis=0)

    eq0 = (ii == i1).astype(jnp.int32)
    eq1 = (ii == i2).astype(jnp.int32)
    c0 = _cumsum_rows(eq0)
    c1 = _cumsum_rows(eq1)
    cnt0 = c0[-1:, :]
    cnt = cnt0 + c1[-1:, :]
    padded = ((cnt + (blk - 1)) // blk) * blk
    lt = (lax.broadcasted_iota(jnp.int32, (n_exp, n_exp), 0)
          < lax.broadcasted_iota(jnp.int32, (n_exp, n_exp), 1)
          ).astype(jnp.float32)
    offs = lax.dot_general(padded.astype(jnp.float32), lt,
                           (((1,), (0,)), ((), ())),
                           preferred_element_type=jnp.float32).astype(jnp.int32)
    pos0_ref[...] = jnp.sum(eq0 * (offs + c0 - 1), axis=1, keepdims=True)
    pos1_ref[...] = jnp.sum(eq1 * (offs + cnt0 + c1 - 1), axis=1, keepdims=True)

    ends = (offs + padded).astype(jnp.int32)
    bstart = lax.broadcasted_iota(jnp.int32, (nblk, 1), 0) * blk
    bexp = jnp.sum((bstart >= ends).astype(jnp.int32), axis=1, keepdims=True)
    bexp_ref[...] = jnp.minimum(bexp, n_exp - 1)
    nact_ref[...] = jnp.sum(padded, axis=1, keepdims=True) // blk


def _router_meta(x32, gate_w, nblk):
    t, _ = x32.shape
    n_exp = gate_w.shape[0]
    return pl.pallas_call(
        functools.partial(_router_meta_body, n_exp=n_exp, blk=_BLK, nblk=nblk),
        out_shape=[
            jax.ShapeDtypeStruct((2 * t, 128), jnp.float32),
            jax.ShapeDtypeStruct((t, 1), jnp.int32),
            jax.ShapeDtypeStruct((t, 1), jnp.int32),
            jax.ShapeDtypeStruct((nblk, 1), jnp.int32),
            jax.ShapeDtypeStruct((1, 1), jnp.int32),
        ],
    )(x32, gate_w)


def _sc_scatter_rows(xp, pos2, tw16, npad):
    t, w = xp.shape
    nw = 32
    cpw = t // nw
    mesh = plsc.VectorSubcoreMesh(core_axis_name="c", subcore_axis_name="s")

    @functools.partial(
        pl.kernel, mesh=mesh,
        out_type=[
            jax.ShapeDtypeStruct((npad, w), jnp.float32),
            jax.ShapeDtypeStruct((npad, 128), jnp.float32),
        ],
        scratch_types=[
            pltpu.VMEM((cpw,), jnp.int32),
            pltpu.VMEM((cpw, w), jnp.float32),
            pltpu.VMEM((cpw, 128), jnp.float32),
            pltpu.SemaphoreType.DMA,
        ],
    )
    def k(x_hbm, pos_hbm, tw_hbm, xs_hbm, cf_hbm, idx_v, rows_v, tw_v, sem):
        wid = lax.axis_index("s") * 2 + lax.axis_index("c")
        base = wid * cpw
        pltpu.sync_copy(x_hbm.at[pl.ds(base, cpw)], rows_v)
        pltpu.sync_copy(pos_hbm.at[0, pl.ds(base, cpw)], idx_v)
        pltpu.async_copy(rows_v, xs_hbm.at[idx_v], sem).wait()
        pltpu.sync_copy(tw_hbm.at[pl.ds(base, cpw)], tw_v)
        pltpu.async_copy(tw_v, cf_hbm.at[idx_v], sem).wait()
        pltpu.sync_copy(pos_hbm.at[1, pl.ds(base, cpw)], idx_v)
        pltpu.async_copy(rows_v, xs_hbm.at[idx_v], sem).wait()
        pltpu.sync_copy(tw_hbm.at[pl.ds(t + base, cpw)], tw_v)
        pltpu.async_copy(tw_v, cf_hbm.at[idx_v], sem).wait()

    return k(xp, pos2, tw16)


def _sc_gather_combine(ys, pos2):
    _, d = ys.shape
    _, t = pos2.shape
    nw = 32
    cpw = t // nw
    nlane = 16
    mesh = plsc.VectorSubcoreMesh(core_axis_name="c", subcore_axis_name="s")

    cph = cpw // 2

    @functools.partial(
        pl.kernel, mesh=mesh,
        out_type=jax.ShapeDtypeStruct((t, d), jnp.float32),
        scratch_types=[
            pltpu.VMEM((cph,), jnp.int32),
            pltpu.VMEM((cph,), jnp.int32),
            pltpu.VMEM((cph, d), jnp.float32),
            pltpu.VMEM((cph, d), jnp.float32),
            pltpu.SemaphoreType.DMA,
            pltpu.SemaphoreType.DMA,
        ],
    )
    def k(ys_hbm, pos_hbm, o_hbm, idx_v, idx2_v, rows_v, rows2_v, sem, sem2):
        wid = lax.axis_index("s") * 2 + lax.axis_index("c")
        base = wid * cpw

        def row_add(r, carry):
            for c in range(d // nlane):
                sl = pl.ds(c * nlane, nlane)
                rows_v[r, sl] = rows_v[r, sl] + rows2_v[r, sl]
            return carry

        for h in range(2):
            off = base + h * cph
            pltpu.sync_copy(pos_hbm.at[0, pl.ds(off, cph)], idx_v)
            cp1 = pltpu.async_copy(ys_hbm.at[idx_v], rows_v, sem)
            pltpu.sync_copy(pos_hbm.at[1, pl.ds(off, cph)], idx2_v)
            cp2 = pltpu.async_copy(ys_hbm.at[idx2_v], rows2_v, sem2)
            cp1.wait()
            cp2.wait()
            lax.fori_loop(0, cph, row_add, 0)
            pltpu.sync_copy(rows_v, o_hbm.at[pl.ds(off, cph)])

    return k(ys, pos2)


def _grouped_glu_body(bexp_s, nact_s, xs_ref, cf_ref, w1_ref, w3_ref, w2_ref,
                      out_ref, acc_ref, *, nf):
    f = pl.program_id(0)
    a = pl.program_id(1)
    na = nact_s[0]

    @pl.when(a < na)
    def _():
        xb = xs_ref[...].astype(jnp.bfloat16)
        w1b = w1_ref[0].astype(jnp.bfloat16)
        w3b = w3_ref[0].astype(jnp.bfloat16)
        t1 = lax.dot_general(xb, w1b, (((1,), (1,)), ((), ())),
                             preferred_element_type=jnp.float32)
        t3 = lax.dot_general(xb, w3b, (((1,), (1,)), ((), ())),
                             preferred_element_type=jnp.float32)
        h = (t1 * jax.nn.sigmoid(t1)) * t3
        w2b = w2_ref[0].astype(jnp.bfloat16)
        o = lax.dot_general(h.astype(jnp.bfloat16), w2b,
                            (((1,), (1,)), ((), ())),
                            preferred_element_type=jnp.float32)

        @pl.when(f == 0)
        def _():
            acc_ref[a] = o

        @pl.when(jnp.logical_and(f != 0, f != nf - 1))
        def _():
            acc_ref[a] += o

        @pl.when(f == nf - 1)
        def _():
            out_ref[...] = (acc_ref[a] + o) * cf_ref[...][:, 0:1]


def _grouped_glu(xs, cf, w1, w3, w2, bexp, nact, f_tile):
    npad, d = xs.shape
    n_exp, dff, _ = w1.shape
    nf = dff // f_tile
    nblk = npad // _BLK

    def amap(a, nact_s):
        return jnp.minimum(a, nact_s[0] - 1)

    grid_spec = pltpu.PrefetchScalarGridSpec(
        num_scalar_prefetch=2,
        grid=(nf, nblk),
        in_specs=[
            pl.BlockSpec((_BLK, d), lambda f, a, be, na: (amap(a, na), 0)),
            pl.BlockSpec((_BLK, 128), lambda f, a, be, na: (amap(a, na), 0)),
            pl.BlockSpec((1, f_tile, d),
                         lambda f, a, be, na: (be[amap(a, na)], f, 0)),
            pl.BlockSpec((1, f_tile, d),
                         lambda f, a, be, na: (be[amap(a, na)], f, 0)),
            pl.BlockSpec((1, d, f_tile),
                         lambda f, a, be, na: (be[amap(a, na)], 0, f)),
        ],
        out_specs=pl.BlockSpec(
            (_BLK, d),
            lambda f, a, be, na: (jnp.where(f == nf - 1, amap(a, na), 0), 0)),
        scratch_shapes=[pltpu.VMEM((nblk, _BLK, d), jnp.float32)],
    )
    return pl.pallas_call(
        functools.partial(_grouped_glu_body, nf=nf),
        grid_spec=grid_spec,
        out_shape=jax.ShapeDtypeStruct((npad, d), jnp.float32),
        compiler_params=pltpu.CompilerParams(
            dimension_semantics=("arbitrary", "arbitrary")),
    )(bexp, nact, xs, cf, w1, w3, w2)


def _moe_pipeline(x32, gate_w, w1, w3, w2, f_tile=512):
    t, d = x32.shape
    n_exp = gate_w.shape[0]
    nblk = (2 * t + (n_exp - 1) * _BLK) // _BLK
    npad = nblk * _BLK
    tw16, pos0, pos1, bexp2, nact2 = _router_meta(x32, gate_w, nblk)
    pos2 = jnp.stack([pos0[:, 0], pos1[:, 0]])
    xs, cf = _sc_scatter_rows(x32, pos2, tw16, npad)
    ys = _grouped_glu(xs, cf, w1, w3, w2, bexp2[:, 0], nact2[0], f_tile)
    return _sc_gather_combine(ys, pos2)


def kernel(hidden_states, gate_w, w1, w3, w2):
    b, s, d = hidden_states.shape
    x32 = hidden_states.reshape(b * s, d)
    out = _moe_pipeline(x32, gate_w, w1, w3, w2, f_tile=896)
    return out.reshape(b, s, d)
